# baseline (speedup 1.0000x reference)
import functools

import jax
import jax.numpy as jnp
from jax import lax
from jax.experimental import pallas as pl
from jax.experimental.pallas import tpu as pltpu
from jax.experimental.pallas import tpu_sc as plsc

BATCH = 16384
FEATS = 20
DIM = 16

NC, NS = 2, 16
NW = NC * NS
B_PER_W = BATCH // NW
ROWS_PER_W = B_PER_W * FEATS
IDXW = 128
IDX_ROWS = ROWS_PER_W // IDXW

CHUNK_B = 64
CHUNK_R = CHUNK_B * FEATS
NG = CHUNK_R // IDXW
NCHUNK = B_PER_W // CHUNK_B


def _lane_sum(v, perms):
    dnums = lax.GatherDimensionNumbers(
        offset_dims=(), collapsed_slice_dims=(0,), start_index_map=(0,))
    for p in perms:
        v = v + lax.gather(
            v, p[:, None], dimension_numbers=dnums, slice_sizes=(1,),
            mode=lax.GatherScatterMode.PROMISE_IN_BOUNDS)
    return v


def _vrsqrt(x):
    i = plsc.bitcast(x, jnp.int32)
    i = jnp.int32(0x5F3759DF) - (i >> 1)
    y = plsc.bitcast(i, jnp.float32)
    for _ in range(3):
        y = y * (1.5 - 0.5 * x * y * y)
    return y


@functools.partial(
    pl.kernel,
    mesh=plsc.VectorSubcoreMesh(core_axis_name="c", subcore_axis_name="s"),
    out_type=jax.ShapeDtypeStruct((BATCH, DIM), jnp.float32),
    compiler_params=pltpu.CompilerParams(
        needs_layout_passes=False, use_tc_tiling_on_sc=False),
    scratch_types=[
        pltpu.VMEM((IDX_ROWS, IDXW), jnp.int32),
        pltpu.VMEM((CHUNK_R, DIM), jnp.float32),
        pltpu.VMEM((B_PER_W, DIM), jnp.float32),
        pltpu.SemaphoreType.DMA,
    ],
)
def _encode(idx_hbm, table_hbm, out_hbm, idx_v, rows_v, out_v, sem):
    wid = lax.axis_index("s") * NC + lax.axis_index("c")
    pltpu.sync_copy(idx_hbm.at[wid], idx_v)
    iota = lax.iota(jnp.int32, DIM)
    perms = [iota ^ s for s in (8, 4, 2, 1)]

    def chunk_body(c, _):
        copies = [
            pltpu.async_copy(
                table_hbm.at[idx_v.at[c * NG + g]],
                rows_v.at[pl.ds(g * IDXW, IDXW)],
                sem,
            )
            for g in range(NG)
        ]
        for cp in copies:
            cp.wait()

        def point_body(bb, _):
            acc = jnp.zeros((DIM,), jnp.float32)
            for j in range(FEATS):
                r = rows_v[bb * FEATS + j]
                acc = acc + r * _vrsqrt(_lane_sum(r * r, perms))
            out_v[c * CHUNK_B + bb] = acc * _vrsqrt(_lane_sum(acc * acc, perms))
            return 0

        lax.fori_loop(0, CHUNK_B, point_body, 0)
        return 0

    lax.fori_loop(0, NCHUNK, chunk_body, 0)
    pltpu.sync_copy(out_v, out_hbm.at[pl.ds(wid * B_PER_W, B_PER_W)])


def kernel(indices, table):
    idx = indices.astype(jnp.int32).reshape(NW, IDX_ROWS, IDXW)
    return _encode(idx, table)
